# baseline (speedup 1.0000x reference)
import math
import functools

import jax
import jax.numpy as jnp
from jax import lax
from jax.experimental import pallas as pl
from jax.experimental.pallas import tpu as pltpu
from jax.experimental.pallas import tpu_sc as plsc

N = 10000
E = 320000
HID = 128
NSF = 6
EMB = HID - NSF
L = 3
BN_EPS = 1e-5

NC = 2
NS = 16
NW = NC * NS
CH = 128

NP = 10240
ROWS_PER_TILE = NP // NS
NCHUNK = 80
EPW = NCHUNK * CH
EP = EPW * NW

_mesh = plsc.VectorSubcoreMesh(
    core_axis_name="c", subcore_axis_name="s", num_cores=NC, num_subcores=NS)


@functools.partial(
    pl.kernel,
    out_type=jax.ShapeDtypeStruct((NC, NP), jnp.float32),
    mesh=_mesh,
    scratch_types=[
        pltpu.VMEM_SHARED((NP,), jnp.float32),
        pltpu.VMEM((NCHUNK, CH), jnp.int32),
        pltpu.VMEM((CH,), jnp.float32),
    ],
)
def _sc_degree(col_hbm, zeros_hbm, out_hbm, acc_sh, cidx_all, ones_v):
    cid = lax.axis_index("c")
    sid = lax.axis_index("s")
    wid = cid * NS + sid
    pltpu.sync_copy(col_hbm.at[wid], cidx_all)
    pltpu.sync_copy(zeros_hbm.at[pl.ds(sid * ROWS_PER_TILE, ROWS_PER_TILE)],
                    acc_sh.at[pl.ds(sid * ROWS_PER_TILE, ROWS_PER_TILE)])
    for j in range(CH // 16):
        ones_v[pl.ds(j * 16, 16)] = jnp.ones((16,), jnp.float32)
    plsc.subcore_barrier()

    def body(i, carry):
        pltpu.sync_copy(ones_v, acc_sh.at[cidx_all.at[i]], add=True)
        return carry

    lax.fori_loop(0, NCHUNK, body, 0)
    plsc.subcore_barrier()
    pltpu.sync_copy(acc_sh.at[pl.ds(sid * ROWS_PER_TILE, ROWS_PER_TILE)],
                    out_hbm.at[cid, pl.ds(sid * ROWS_PER_TILE, ROWS_PER_TILE)])


@functools.partial(
    pl.kernel,
    out_type=jax.ShapeDtypeStruct((NC, NP, HID), jnp.float32),
    mesh=_mesh,
    scratch_types=[
        pltpu.VMEM_SHARED((NP, HID), jnp.float32),
        pltpu.VMEM((NCHUNK // 2, CH), jnp.int32),
        pltpu.VMEM((NCHUNK // 2, CH), jnp.int32),
        [pltpu.VMEM((CH, HID), jnp.float32) for _ in range(2)],
        [pltpu.SemaphoreType.DMA for _ in range(2)],
    ],
)
def _sc_aggregate(hp_hbm, row_hbm, col_hbm, out_hbm,
                  acc_sh, ridx_all, cidx_all, bufs, gsems):
    IH = NCHUNK // 2
    cid = lax.axis_index("c")
    sid = lax.axis_index("s")
    wid = cid * NS + sid
    pltpu.sync_copy(row_hbm.at[wid, pl.ds(0, NCHUNK // 2)], ridx_all)
    pltpu.sync_copy(col_hbm.at[wid, pl.ds(0, NCHUNK // 2)], cidx_all)
    pltpu.async_copy(hp_hbm.at[ridx_all.at[1]], bufs[1], gsems[1])
    for r in range(CH):
        for j in range(HID // 16):
            bufs[0][r, pl.ds(j * 16, 16)] = jnp.zeros((16,), jnp.float32)
    for z in range(ROWS_PER_TILE // CH):
        pltpu.sync_copy(bufs[0],
                        acc_sh.at[pl.ds(sid * ROWS_PER_TILE + z * CH, CH)])
    pltpu.async_copy(hp_hbm.at[ridx_all.at[0]], bufs[0], gsems[0])
    plsc.subcore_barrier()

    def body(g, carry):
        for b in range(2):
            i = 2 * g + b
            pltpu.make_async_copy(hp_hbm.at[ridx_all.at[i]], bufs[b],
                                  gsems[b]).wait()
            pltpu.sync_copy(bufs[b], acc_sh.at[cidx_all.at[i]], add=True)

            @pl.when(i + 2 < IH)
            def _():
                pltpu.async_copy(hp_hbm.at[ridx_all.at[i + 2]], bufs[b],
                                 gsems[b])
        return carry

    lax.fori_loop(0, IH // 2, body, 0)
    pltpu.sync_copy(row_hbm.at[wid, pl.ds(IH, IH)], ridx_all)
    pltpu.sync_copy(col_hbm.at[wid, pl.ds(IH, IH)], cidx_all)
    for b in range(2):
        pltpu.async_copy(hp_hbm.at[ridx_all.at[b]], bufs[b], gsems[b])
    lax.fori_loop(0, IH // 2, body, 0)
    plsc.subcore_barrier()
    pltpu.sync_copy(acc_sh.at[pl.ds(sid * ROWS_PER_TILE, ROWS_PER_TILE)],
                    out_hbm.at[cid, pl.ds(sid * ROWS_PER_TILE, ROWS_PER_TILE)])


BR = 1024
GRID = NP // BR


def _scale(m, dis8):
    t = jnp.transpose(dis8)
    return (m.reshape(HID, 8, HID) * t[:, :, None]).reshape(BR, HID)


def _k0a_body(emb_ref, ns_ref, fpw_ref, fpb_ref, w0a_ref, w0b_ref, h_ref):
    struct = jnp.maximum(
        jnp.dot(ns_ref[...], fpw_ref[...],
                preferred_element_type=jnp.float32) + fpb_ref[...], 0.0)
    h_ref[...] = (
        jnp.dot(emb_ref[...], w0a_ref[...], preferred_element_type=jnp.float32)
        + jnp.dot(struct, w0b_ref[...], preferred_element_type=jnp.float32))


def _k0b_body(h_ref, deg_ref, hp_ref, dis_ref):
    d = deg_ref[0] + deg_ref[1]
    dis8 = jnp.where(d > 0.0,
                     lax.rsqrt(jnp.maximum(d, 1e-12)),
                     jnp.zeros_like(d))
    hp_ref[...] = _scale(h_ref[...], dis8)
    dis_ref[...] = dis8


def _k1_body(p_ref, dis_ref, a_ref, c_ref, w_ref, x_out_ref, hp_ref):
    dis8 = dis_ref[...]
    agg = _scale(p_ref[0] + p_ref[1], dis8)
    x = jnp.maximum(agg * a_ref[...] + c_ref[...], 0.0)
    x_out_ref[...] = x
    hp_ref[...] = _scale(
        jnp.dot(x, w_ref[...], preferred_element_type=jnp.float32), dis8)


def _k2_body(p_ref, x_ref, dis_ref, a_ref, c_ref, w_ref, x_out_ref, hp_ref):
    dis8 = dis_ref[...]
    agg = _scale(p_ref[0] + p_ref[1], dis8)
    x = jnp.maximum(agg * a_ref[...] + c_ref[...], 0.0) + 0.5 * x_ref[...]
    x_out_ref[...] = x
    hp_ref[...] = _scale(
        jnp.dot(x, w_ref[...], preferred_element_type=jnp.float32), dis8)


def _k3_body(p_ref, x_ref, dis_ref, a_ref, c_ref, out_ref):
    agg = _scale(p_ref[0] + p_ref[1], dis_ref[...])
    out_ref[...] = (jnp.maximum(agg * a_ref[...] + c_ref[...], 0.0)
                    + 0.5 * x_ref[...])


def _row_spec(width):
    return pl.BlockSpec((BR, width), lambda i: (i, 0))


def _full_spec(shape):
    return pl.BlockSpec(shape, lambda i: tuple(0 for _ in shape))


_P_SPEC = pl.BlockSpec((NC, BR, HID), lambda i: (0, i, 0))
_DEG_SPEC = pl.BlockSpec((NC, BR // HID, HID), lambda i: (0, i, 0))
_DIS_SPEC = pl.BlockSpec((BR // HID, HID), lambda i: (i, 0))
NDB = NP // HID


def _tc_k0a(emb, ns, fp_W, fp_b, w0a, w0b):
    return pl.pallas_call(
        _k0a_body,
        grid=(GRID,),
        in_specs=[
            _row_spec(EMB), _row_spec(NSF),
            _full_spec((NSF, NSF)), _full_spec((1, NSF)),
            _full_spec((EMB, HID)), _full_spec((NSF, HID)),
        ],
        out_specs=_row_spec(HID),
        out_shape=jax.ShapeDtypeStruct((NP, HID), jnp.float32),
    )(emb, ns, fp_W, fp_b, w0a, w0b)


def _tc_k0b(h, degp):
    return pl.pallas_call(
        _k0b_body,
        grid=(GRID,),
        in_specs=[_row_spec(HID), _DEG_SPEC],
        out_specs=[_row_spec(HID), _DIS_SPEC],
        out_shape=[jax.ShapeDtypeStruct((NP, HID), jnp.float32),
                   jax.ShapeDtypeStruct((NDB, HID), jnp.float32)],
    )(h, degp)


def _tc_k1(p, dis, a, c, w):
    return pl.pallas_call(
        _k1_body,
        grid=(GRID,),
        in_specs=[
            _P_SPEC, _DIS_SPEC,
            _full_spec((1, HID)), _full_spec((1, HID)),
            _full_spec((HID, HID)),
        ],
        out_specs=[_row_spec(HID), _row_spec(HID)],
        out_shape=[jax.ShapeDtypeStruct((NP, HID), jnp.float32),
                   jax.ShapeDtypeStruct((NP, HID), jnp.float32)],
    )(p, dis, a, c, w)


def _tc_k2(p, x, dis, a, c, w):
    return pl.pallas_call(
        _k2_body,
        grid=(GRID,),
        in_specs=[
            _P_SPEC, _row_spec(HID), _DIS_SPEC,
            _full_spec((1, HID)), _full_spec((1, HID)),
            _full_spec((HID, HID)),
        ],
        out_specs=[_row_spec(HID), _row_spec(HID)],
        out_shape=[jax.ShapeDtypeStruct((NP, HID), jnp.float32),
                   jax.ShapeDtypeStruct((NP, HID), jnp.float32)],
    )(p, x, dis, a, c, w)


def _tc_k3(p, x, dis, a, c):
    return pl.pallas_call(
        _k3_body,
        grid=(GRID,),
        in_specs=[
            _P_SPEC, _row_spec(HID), _DIS_SPEC,
            _full_spec((1, HID)), _full_spec((1, HID)),
        ],
        out_specs=_row_spec(HID),
        out_shape=jax.ShapeDtypeStruct((N, HID), jnp.float32),
    )(p, x, dis, a, c)


def kernel(emb_weight, node_struct, fp_W, fp_b, conv_W, conv_b, bn_gamma,
           bn_beta, edge_index):
    f32 = jnp.float32
    row = edge_index[0].astype(jnp.int32)
    col = edge_index[1].astype(jnp.int32)
    pad = EP - E
    pad_ar = jnp.arange(pad, dtype=jnp.int32)
    row_p = jnp.concatenate([row, pad_ar % N]).reshape(NW, NCHUNK, CH)
    col_p = jnp.concatenate(
        [col, N + pad_ar % (NP - N)]).reshape(NW, NCHUNK, CH)

    zeros_deg = jnp.zeros((NP,), f32)

    s = 1.0 / math.sqrt(1.0 + BN_EPS)
    A = bn_gamma * s
    Cv = conv_b * A + bn_beta

    colf = col_p.reshape(-1)
    col_t = (((colf // BR) * 8 + colf % 8) * HID
             + (colf % BR) // 8).reshape(NW, NCHUNK, CH)
    degp = _sc_degree(col_t, zeros_deg)
    degp3 = degp.reshape(NC, NDB, HID)

    w0a = conv_W[0, :EMB, :]
    w0b = conv_W[0, EMB:, :]
    h0 = _tc_k0a(emb_weight, node_struct, fp_W, fp_b.reshape(1, NSF),
                 w0a, w0b)
    hp, dis = _tc_k0b(h0, degp3)

    p0 = _sc_aggregate(hp, row_p, col_p)
    x1, hp = _tc_k1(p0, dis, A[0].reshape(1, HID), Cv[0].reshape(1, HID),
                    conv_W[1])

    p1 = _sc_aggregate(hp, row_p, col_p)
    x2, hp = _tc_k2(p1, x1, dis, A[1].reshape(1, HID), Cv[1].reshape(1, HID),
                    conv_W[2])

    p2 = _sc_aggregate(hp, row_p, col_p)
    return _tc_k3(p2, x2, dis, A[2].reshape(1, HID), Cv[2].reshape(1, HID))


# speedup vs baseline: 1.0066x; 1.0066x over previous
import math
import functools

import jax
import jax.numpy as jnp
from jax import lax
from jax.experimental import pallas as pl
from jax.experimental.pallas import tpu as pltpu
from jax.experimental.pallas import tpu_sc as plsc

N = 10000
E = 320000
HID = 128
NSF = 6
EMB = HID - NSF
L = 3
BN_EPS = 1e-5

NC = 2
NS = 16
NW = NC * NS
CH = 128

NP = 10240
ROWS_PER_TILE = NP // NS
NCHUNK = 80
EPW = NCHUNK * CH
EP = EPW * NW

_mesh = plsc.VectorSubcoreMesh(
    core_axis_name="c", subcore_axis_name="s", num_cores=NC, num_subcores=NS)


@functools.partial(
    pl.kernel,
    out_type=jax.ShapeDtypeStruct((NC, NP), jnp.float32),
    mesh=_mesh,
    scratch_types=[
        pltpu.VMEM_SHARED((NP,), jnp.float32),
        pltpu.VMEM((NCHUNK, CH), jnp.int32),
        pltpu.VMEM((CH,), jnp.float32),
    ],
)
def _sc_degree(col_hbm, zeros_hbm, out_hbm, acc_sh, cidx_all, ones_v):
    cid = lax.axis_index("c")
    sid = lax.axis_index("s")
    wid = cid * NS + sid
    pltpu.sync_copy(col_hbm.at[wid], cidx_all)
    pltpu.sync_copy(zeros_hbm.at[pl.ds(sid * ROWS_PER_TILE, ROWS_PER_TILE)],
                    acc_sh.at[pl.ds(sid * ROWS_PER_TILE, ROWS_PER_TILE)])
    for j in range(CH // 16):
        ones_v[pl.ds(j * 16, 16)] = jnp.ones((16,), jnp.float32)
    plsc.subcore_barrier()

    def body(i, carry):
        pltpu.sync_copy(ones_v, acc_sh.at[cidx_all.at[i]], add=True)
        return carry

    lax.fori_loop(0, NCHUNK, body, 0)
    plsc.subcore_barrier()
    pltpu.sync_copy(acc_sh.at[pl.ds(sid * ROWS_PER_TILE, ROWS_PER_TILE)],
                    out_hbm.at[cid, pl.ds(sid * ROWS_PER_TILE, ROWS_PER_TILE)])


@functools.partial(
    pl.kernel,
    out_type=jax.ShapeDtypeStruct((NC, NP, HID), jnp.float32),
    mesh=_mesh,
    scratch_types=[
        pltpu.VMEM_SHARED((NP, HID), jnp.float32),
        pltpu.VMEM((NCHUNK // 2, CH), jnp.int32),
        pltpu.VMEM((NCHUNK // 2, CH), jnp.int32),
        [pltpu.VMEM((CH, HID), jnp.float32) for _ in range(2)],
        [pltpu.SemaphoreType.DMA for _ in range(2)],
    ],
)
def _sc_aggregate(hp_hbm, row_hbm, col_hbm, out_hbm,
                  acc_sh, ridx_all, cidx_all, bufs, gsems):
    IH = NCHUNK // 2
    cid = lax.axis_index("c")
    sid = lax.axis_index("s")
    wid = cid * NS + sid
    pltpu.sync_copy(row_hbm.at[wid, pl.ds(0, NCHUNK // 2)], ridx_all)
    pltpu.sync_copy(col_hbm.at[wid, pl.ds(0, NCHUNK // 2)], cidx_all)
    pltpu.async_copy(hp_hbm.at[ridx_all.at[1]], bufs[1], gsems[1])
    for r in range(CH):
        for j in range(HID // 16):
            bufs[0][r, pl.ds(j * 16, 16)] = jnp.zeros((16,), jnp.float32)
    for z in range(ROWS_PER_TILE // CH):
        pltpu.sync_copy(bufs[0],
                        acc_sh.at[pl.ds(sid * ROWS_PER_TILE + z * CH, CH)])
    pltpu.async_copy(hp_hbm.at[ridx_all.at[0]], bufs[0], gsems[0])
    plsc.subcore_barrier()

    def body(g, carry):
        for b in range(2):
            i = 2 * g + b
            pltpu.make_async_copy(hp_hbm.at[ridx_all.at[i]], bufs[b],
                                  gsems[b]).wait()
            pltpu.sync_copy(bufs[b], acc_sh.at[cidx_all.at[i]], add=True)

            @pl.when(i + 2 < IH)
            def _():
                pltpu.async_copy(hp_hbm.at[ridx_all.at[i + 2]], bufs[b],
                                 gsems[b])
        return carry

    lax.fori_loop(0, IH // 2, body, 0)
    pltpu.sync_copy(row_hbm.at[wid, pl.ds(IH, IH)], ridx_all)
    pltpu.sync_copy(col_hbm.at[wid, pl.ds(IH, IH)], cidx_all)
    for b in range(2):
        pltpu.async_copy(hp_hbm.at[ridx_all.at[b]], bufs[b], gsems[b])
    lax.fori_loop(0, IH // 2, body, 0)
    plsc.subcore_barrier()
    pltpu.sync_copy(acc_sh.at[pl.ds(sid * ROWS_PER_TILE, ROWS_PER_TILE)],
                    out_hbm.at[cid, pl.ds(sid * ROWS_PER_TILE, ROWS_PER_TILE)])


BR = 1024
GRID = NP // BR


def _scale(m, dis8):
    t = jnp.transpose(dis8)
    return (m.reshape(HID, 8, HID) * t[:, :, None]).reshape(BR, HID)


def _k0_body(emb_ref, ns_ref, fpw_ref, fpb_ref, w0a_ref, w0b_ref, deg_ref,
             hp_ref, dis_ref):
    d = deg_ref[0] + deg_ref[1]
    dis8 = jnp.where(d > 0.0,
                     lax.rsqrt(jnp.maximum(d, 1e-12)),
                     jnp.zeros_like(d))
    struct = jnp.maximum(
        jnp.dot(ns_ref[...], fpw_ref[...],
                preferred_element_type=jnp.float32) + fpb_ref[...], 0.0)
    h = (jnp.dot(emb_ref[...], w0a_ref[...], preferred_element_type=jnp.float32)
         + jnp.dot(struct, w0b_ref[...], preferred_element_type=jnp.float32))
    hp_ref[...] = _scale(h, dis8)
    dis_ref[...] = dis8


def _k1_body(p_ref, dis_ref, a_ref, c_ref, w_ref, x_out_ref, hp_ref):
    dis8 = dis_ref[...]
    agg = _scale(p_ref[0] + p_ref[1], dis8)
    x = jnp.maximum(agg * a_ref[...] + c_ref[...], 0.0)
    x_out_ref[...] = x
    hp_ref[...] = _scale(
        jnp.dot(x, w_ref[...], preferred_element_type=jnp.float32), dis8)


def _k2_body(p_ref, x_ref, dis_ref, a_ref, c_ref, w_ref, x_out_ref, hp_ref):
    dis8 = dis_ref[...]
    agg = _scale(p_ref[0] + p_ref[1], dis8)
    x = jnp.maximum(agg * a_ref[...] + c_ref[...], 0.0) + 0.5 * x_ref[...]
    x_out_ref[...] = x
    hp_ref[...] = _scale(
        jnp.dot(x, w_ref[...], preferred_element_type=jnp.float32), dis8)


def _k3_body(p_ref, x_ref, dis_ref, a_ref, c_ref, out_ref):
    agg = _scale(p_ref[0] + p_ref[1], dis_ref[...])
    out_ref[...] = (jnp.maximum(agg * a_ref[...] + c_ref[...], 0.0)
                    + 0.5 * x_ref[...])


def _row_spec(width):
    return pl.BlockSpec((BR, width), lambda i: (i, 0))


def _full_spec(shape):
    return pl.BlockSpec(shape, lambda i: tuple(0 for _ in shape))


_P_SPEC = pl.BlockSpec((NC, BR, HID), lambda i: (0, i, 0))
_DEG_SPEC = pl.BlockSpec((NC, BR // HID, HID), lambda i: (0, i, 0))
_DIS_SPEC = pl.BlockSpec((BR // HID, HID), lambda i: (i, 0))
NDB = NP // HID


def _tc_k0(emb, ns, fp_W, fp_b, w0a, w0b, degp):
    return pl.pallas_call(
        _k0_body,
        grid=(GRID,),
        in_specs=[
            _row_spec(EMB), _row_spec(NSF),
            _full_spec((NSF, NSF)), _full_spec((1, NSF)),
            _full_spec((EMB, HID)), _full_spec((NSF, HID)),
            _DEG_SPEC,
        ],
        out_specs=[_row_spec(HID), _DIS_SPEC],
        out_shape=[jax.ShapeDtypeStruct((NP, HID), jnp.float32),
                   jax.ShapeDtypeStruct((NDB, HID), jnp.float32)],
    )(emb, ns, fp_W, fp_b, w0a, w0b, degp)


def _tc_k1(p, dis, a, c, w):
    return pl.pallas_call(
        _k1_body,
        grid=(GRID,),
        in_specs=[
            _P_SPEC, _DIS_SPEC,
            _full_spec((1, HID)), _full_spec((1, HID)),
            _full_spec((HID, HID)),
        ],
        out_specs=[_row_spec(HID), _row_spec(HID)],
        out_shape=[jax.ShapeDtypeStruct((NP, HID), jnp.float32),
                   jax.ShapeDtypeStruct((NP, HID), jnp.float32)],
    )(p, dis, a, c, w)


def _tc_k2(p, x, dis, a, c, w):
    return pl.pallas_call(
        _k2_body,
        grid=(GRID,),
        in_specs=[
            _P_SPEC, _row_spec(HID), _DIS_SPEC,
            _full_spec((1, HID)), _full_spec((1, HID)),
            _full_spec((HID, HID)),
        ],
        out_specs=[_row_spec(HID), _row_spec(HID)],
        out_shape=[jax.ShapeDtypeStruct((NP, HID), jnp.float32),
                   jax.ShapeDtypeStruct((NP, HID), jnp.float32)],
    )(p, x, dis, a, c, w)


def _tc_k3(p, x, dis, a, c):
    return pl.pallas_call(
        _k3_body,
        grid=(GRID,),
        in_specs=[
            _P_SPEC, _row_spec(HID), _DIS_SPEC,
            _full_spec((1, HID)), _full_spec((1, HID)),
        ],
        out_specs=_row_spec(HID),
        out_shape=jax.ShapeDtypeStruct((N, HID), jnp.float32),
    )(p, x, dis, a, c)


def kernel(emb_weight, node_struct, fp_W, fp_b, conv_W, conv_b, bn_gamma,
           bn_beta, edge_index):
    f32 = jnp.float32
    row = edge_index[0].astype(jnp.int32)
    col = edge_index[1].astype(jnp.int32)
    pad = EP - E
    pad_ar = jnp.arange(pad, dtype=jnp.int32)
    row_p = jnp.concatenate([row, pad_ar % N]).reshape(NW, NCHUNK, CH)
    col_p = jnp.concatenate(
        [col, N + pad_ar % (NP - N)]).reshape(NW, NCHUNK, CH)

    zeros_deg = jnp.zeros((NP,), f32)

    s = 1.0 / math.sqrt(1.0 + BN_EPS)
    A = bn_gamma * s
    Cv = conv_b * A + bn_beta

    colf = col_p.reshape(-1)
    col_t = (((colf // BR) * 8 + colf % 8) * HID
             + (colf % BR) // 8).reshape(NW, NCHUNK, CH)
    degp = _sc_degree(col_t, zeros_deg)
    degp3 = degp.reshape(NC, NDB, HID)

    w0a = conv_W[0, :EMB, :]
    w0b = conv_W[0, EMB:, :]
    hp, dis = _tc_k0(emb_weight, node_struct, fp_W, fp_b.reshape(1, NSF),
                     w0a, w0b, degp3)

    p0 = _sc_aggregate(hp, row_p, col_p)
    x1, hp = _tc_k1(p0, dis, A[0].reshape(1, HID), Cv[0].reshape(1, HID),
                    conv_W[1])

    p1 = _sc_aggregate(hp, row_p, col_p)
    x2, hp = _tc_k2(p1, x1, dis, A[1].reshape(1, HID), Cv[1].reshape(1, HID),
                    conv_W[2])

    p2 = _sc_aggregate(hp, row_p, col_p)
    return _tc_k3(p2, x2, dis, A[2].reshape(1, HID), Cv[2].reshape(1, HID))
